# baseline (speedup 1.0000x reference)
import functools

import jax
import jax.numpy as jnp
from jax import lax
from jax.experimental import pallas as pl
from jax.experimental.pallas import tpu as pltpu
from jax.experimental.pallas import tpu_sc as plsc

NUM_CODES = 8192
DIM = 256
TOKENS = 16384

TN = 512
TK = 512


def _argmin_body(flat_ref, x2_ref, w2_ref, w_ref, idx_ref):
    flat = flat_ref[...].astype(jnp.bfloat16)
    x2 = x2_ref[...]
    lanes = lax.broadcasted_iota(jnp.int32, (TN, TK), 1)

    def step(kt, carry):
        best, bidx = carry
        wt = w_ref[pl.ds(kt * TK, TK), :].astype(jnp.bfloat16)
        ab = lax.dot_general(flat, wt, (((1,), (1,)), ((), ())),
                             preferred_element_type=jnp.float32)
        w2t = w2_ref[:, pl.ds(kt * TK, TK)]
        dist = (x2 - 2.0 * ab) + w2t
        upd = dist < best
        return (jnp.where(upd, dist, best),
                jnp.where(upd, lanes + kt * TK, bidx))

    best0 = jnp.full((TN, TK), jnp.inf, jnp.float32)
    bidx0 = jnp.zeros((TN, TK), jnp.int32)
    best, bidx = lax.fori_loop(0, NUM_CODES // TK, step, (best0, bidx0))
    m = jnp.min(best, axis=1, keepdims=True)
    cand = jnp.where(best == m, bidx, NUM_CODES)
    idx_ref[...] = jnp.min(cand, axis=1, keepdims=True)


def _argmin_call(flat, x2, w2, W):
    return pl.pallas_call(
        _argmin_body,
        grid=(TOKENS // TN,),
        in_specs=[
            pl.BlockSpec((TN, DIM), lambda n: (n, 0)),
            pl.BlockSpec((TN, 1), lambda n: (n, 0)),
            pl.BlockSpec((1, NUM_CODES), lambda n: (0, 0)),
            pl.BlockSpec((NUM_CODES, DIM), lambda n: (0, 0)),
        ],
        out_specs=pl.BlockSpec((TN, 1), lambda n: (n, 0)),
        out_shape=jax.ShapeDtypeStruct((TOKENS, 1), jnp.int32),
    )(flat, x2, w2, W)


def _make_sc_gather():
    info = plsc.get_sparse_core_info()
    nw = info.num_cores * info.num_subcores
    b_per_w = TOKENS // nw
    ch = 128
    nch = b_per_w // ch
    mesh = plsc.VectorSubcoreMesh(core_axis_name="c", subcore_axis_name="s")

    @functools.partial(
        pl.kernel, mesh=mesh,
        out_type=jax.ShapeDtypeStruct((TOKENS, DIM), jnp.float32),
        scratch_types=[
            pltpu.VMEM((b_per_w,), jnp.int32),
            pltpu.VMEM((ch, DIM), jnp.float32),
            pltpu.VMEM((ch, DIM), jnp.float32),
            pltpu.SemaphoreType.DMA,
            pltpu.SemaphoreType.DMA,
        ],
    )
    def gather(table_hbm, idx_hbm, out_hbm, idx_v, rows_a, rows_b, sem_a, sem_b):
        wid = lax.axis_index("s") * info.num_cores + lax.axis_index("c")
        base = wid * b_per_w
        pltpu.sync_copy(idx_hbm.at[pl.ds(base, b_per_w)], idx_v)
        bufs = (rows_a, rows_b)
        sems = (sem_a, sem_b)
        handles = [None] * nch
        handles[0] = pltpu.async_copy(
            table_hbm.at[idx_v.at[pl.ds(0, ch)]], bufs[0], sems[0])
        for c in range(nch):
            if c + 1 < nch:
                handles[c + 1] = pltpu.async_copy(
                    table_hbm.at[idx_v.at[pl.ds((c + 1) * ch, ch)]],
                    bufs[(c + 1) % 2], sems[(c + 1) % 2])
            handles[c].wait()
            pltpu.sync_copy(bufs[c % 2], out_hbm.at[pl.ds(base + c * ch, ch)])

    return gather


_sc_gather = None


def _gather_rows(W, idx):
    global _sc_gather
    if _sc_gather is None:
        _sc_gather = _make_sc_gather()
    return _sc_gather(W, idx)


def kernel(latents, W):
    b, d, h, w = latents.shape
    flat = jnp.transpose(latents, (0, 2, 3, 1)).reshape(-1, d)
    x2 = (flat ** 2).sum(axis=-1, keepdims=True)
    w2 = (W ** 2).sum(axis=-1).reshape(1, NUM_CODES)
    idx = _argmin_call(flat, x2, w2, W).reshape(-1)
    quantized = _gather_rows(W, idx)
    quantized_st = (quantized - flat) + flat
    return jnp.transpose(quantized_st.reshape(b, h, w, d), (0, 3, 1, 2))


# speedup vs baseline: 1.0708x; 1.0708x over previous
import functools

import jax
import jax.numpy as jnp
from jax import lax
from jax.experimental import pallas as pl
from jax.experimental.pallas import tpu as pltpu
from jax.experimental.pallas import tpu_sc as plsc

NUM_CODES = 8192
DIM = 256
TOKENS = 16384

TN = 512
TK = 1024


def _argmin_body(flat_ref, x2_ref, w2_ref, w_ref, idx_ref):
    flat = flat_ref[...].astype(jnp.bfloat16)
    x2 = x2_ref[...]
    lanes = lax.broadcasted_iota(jnp.int32, (TN, TK), 1)

    def step(kt, carry):
        best, bidx = carry
        wt = w_ref[pl.ds(kt * TK, TK), :].astype(jnp.bfloat16)
        ab = lax.dot_general(flat, wt, (((1,), (1,)), ((), ())),
                             preferred_element_type=jnp.float32)
        w2t = w2_ref[:, pl.ds(kt * TK, TK)]
        dist = (x2 - 2.0 * ab) + w2t
        upd = dist < best
        return (jnp.where(upd, dist, best),
                jnp.where(upd, lanes + kt * TK, bidx))

    best0 = jnp.full((TN, TK), jnp.inf, jnp.float32)
    bidx0 = jnp.zeros((TN, TK), jnp.int32)
    best, bidx = lax.fori_loop(0, NUM_CODES // TK, step, (best0, bidx0))
    m = jnp.min(best, axis=1, keepdims=True)
    cand = jnp.where(best == m, bidx, NUM_CODES)
    idx_ref[...] = jnp.min(cand, axis=1, keepdims=True)


def _argmin_call(flat, x2, w2, W):
    return pl.pallas_call(
        _argmin_body,
        grid=(TOKENS // TN,),
        in_specs=[
            pl.BlockSpec((TN, DIM), lambda n: (n, 0)),
            pl.BlockSpec((TN, 1), lambda n: (n, 0)),
            pl.BlockSpec((1, NUM_CODES), lambda n: (0, 0)),
            pl.BlockSpec((NUM_CODES, DIM), lambda n: (0, 0)),
        ],
        out_specs=pl.BlockSpec((TN, 1), lambda n: (n, 0)),
        out_shape=jax.ShapeDtypeStruct((TOKENS, 1), jnp.int32),
    )(flat, x2, w2, W)


def _make_sc_gather():
    info = plsc.get_sparse_core_info()
    nw = info.num_cores * info.num_subcores
    b_per_w = TOKENS // nw
    ch = 128
    nch = b_per_w // ch
    mesh = plsc.VectorSubcoreMesh(core_axis_name="c", subcore_axis_name="s")

    @functools.partial(
        pl.kernel, mesh=mesh,
        out_type=jax.ShapeDtypeStruct((TOKENS, DIM), jnp.float32),
        scratch_types=[
            pltpu.VMEM((b_per_w,), jnp.int32),
            pltpu.VMEM((ch, DIM), jnp.float32),
            pltpu.VMEM((ch, DIM), jnp.float32),
            pltpu.SemaphoreType.DMA,
            pltpu.SemaphoreType.DMA,
        ],
    )
    def gather(table_hbm, idx_hbm, out_hbm, idx_v, rows_a, rows_b, sem_a, sem_b):
        wid = lax.axis_index("s") * info.num_cores + lax.axis_index("c")
        base = wid * b_per_w
        pltpu.sync_copy(idx_hbm.at[pl.ds(base, b_per_w)], idx_v)
        bufs = (rows_a, rows_b)
        sems = (sem_a, sem_b)
        handles = [None] * nch
        handles[0] = pltpu.async_copy(
            table_hbm.at[idx_v.at[pl.ds(0, ch)]], bufs[0], sems[0])
        for c in range(nch):
            if c + 1 < nch:
                handles[c + 1] = pltpu.async_copy(
                    table_hbm.at[idx_v.at[pl.ds((c + 1) * ch, ch)]],
                    bufs[(c + 1) % 2], sems[(c + 1) % 2])
            handles[c].wait()
            pltpu.sync_copy(bufs[c % 2], out_hbm.at[pl.ds(base + c * ch, ch)])

    return gather


_sc_gather = None


def _gather_rows(W, idx):
    global _sc_gather
    if _sc_gather is None:
        _sc_gather = _make_sc_gather()
    return _sc_gather(W, idx)


def kernel(latents, W):
    b, d, h, w = latents.shape
    flat = jnp.transpose(latents, (0, 2, 3, 1)).reshape(-1, d)
    x2 = (flat ** 2).sum(axis=-1, keepdims=True)
    w2 = (W ** 2).sum(axis=-1).reshape(1, NUM_CODES)
    idx = _argmin_call(flat, x2, w2, W).reshape(-1)
    quantized = _gather_rows(W, idx)
    quantized_st = (quantized - flat) + flat
    return jnp.transpose(quantized_st.reshape(b, h, w, d), (0, 3, 1, 2))
